# baseline (speedup 1.0000x reference)
import functools

import jax
import jax.numpy as jnp
from jax.experimental import pallas as pl


def _pick_rows(n, target):
    best = 8
    b = 8
    while b <= target:
        if n % b == 0:
            best = b
        b += 8
    return best


def _bdot(a, b):
    return jnp.dot(a.astype(jnp.bfloat16), b.astype(jnp.bfloat16),
                   preferred_element_type=jnp.float32)


def _mm_body(a_ref, h_ref, o_ref):
    o_ref[...] = _bdot(a_ref[...], h_ref[...])


def _adj_matmul(adj, h):
    n, k = adj.shape
    d = h.shape[1]
    bm = _pick_rows(n, 400)
    return pl.pallas_call(
        _mm_body,
        grid=(n // bm,),
        in_specs=[
            pl.BlockSpec((bm, k), lambda i: (i, 0)),
            pl.BlockSpec((k, d), lambda i: (0, 0)),
        ],
        out_specs=pl.BlockSpec((bm, d), lambda i: (i, 0)),
        out_shape=jax.ShapeDtypeStruct((n, d), jnp.float32),
    )(adj, h)


def _row_specs(bm, d, hdim, extra):
    specs = [pl.BlockSpec((bm, d), lambda i: (i, 0)),
             pl.BlockSpec((d, hdim), lambda i: (0, 0)),
             pl.BlockSpec((1, hdim), lambda i: (0, 0))]
    specs += [pl.BlockSpec((1, hdim), lambda i: (0, 0))] * extra
    return specs


def _y_of(p_ref, w_ref, b_ref):
    return _bdot(p_ref[...], w_ref[...]) + b_ref[...]


def _mask8(hdim):
    return jnp.concatenate([jnp.ones((1, hdim), jnp.float32),
                            jnp.zeros((7, hdim), jnp.float32)], axis=0)


def _colsum_y(pooled, w, b):
    n, d = pooled.shape
    hdim = w.shape[1]
    bm = _pick_rows(n, 2000)

    def body(p_ref, w_ref, b_ref, s_ref):
        i = pl.program_id(0)
        y = _y_of(p_ref, w_ref, b_ref)
        blk = jnp.sum(y, axis=0, keepdims=True) * _mask8(hdim)

        @pl.when(i == 0)
        def _():
            s_ref[...] = jnp.zeros_like(s_ref)

        s_ref[...] += blk

    return pl.pallas_call(
        body,
        grid=(n // bm,),
        in_specs=_row_specs(bm, d, hdim, 0),
        out_specs=pl.BlockSpec((8, hdim), lambda i: (0, 0)),
        out_shape=jax.ShapeDtypeStruct((8, hdim), jnp.float32),
    )(pooled, w, b)


def _colsumsq_y(pooled, w, b, ysum):
    n, d = pooled.shape
    hdim = w.shape[1]
    bm = _pick_rows(n, 2000)
    c = 1.0 / n

    def body(p_ref, w_ref, b_ref, s1_ref, s_ref):
        i = pl.program_id(0)
        y = _y_of(p_ref, w_ref, b_ref)
        mean = s1_ref[0:1, :] * c
        dd = y - mean
        blk = jnp.sum(dd * dd, axis=0, keepdims=True) * _mask8(hdim)

        @pl.when(i == 0)
        def _():
            s_ref[...] = jnp.zeros_like(s_ref)

        s_ref[...] += blk

    specs = _row_specs(bm, d, hdim, 0)
    specs.append(pl.BlockSpec((8, hdim), lambda i: (0, 0)))
    return pl.pallas_call(
        body,
        grid=(n // bm,),
        in_specs=specs,
        out_specs=pl.BlockSpec((8, hdim), lambda i: (0, 0)),
        out_shape=jax.ShapeDtypeStruct((8, hdim), jnp.float32),
    )(pooled, w, b, ysum)


def _apply_and_z(pooled, w1, b1, g1, be1, w2, b2, ysum, ysumsq):
    n, d = pooled.shape
    hdim = w1.shape[1]
    bm = _pick_rows(n, 2000)
    c = 1.0 / n

    def body(p_ref, w1_ref, b1_ref, g1_ref, be1_ref, w2_ref, b2_ref,
             s1_ref, s2_ref, z_ref, zs_ref):
        i = pl.program_id(0)
        y = _y_of(p_ref, w1_ref, b1_ref)
        mean = s1_ref[0:1, :] * c
        den = jnp.sqrt(s2_ref[0:1, :] * c + 1e-5)
        h1 = jnp.maximum(g1_ref[...] * (y - mean) / den + be1_ref[...], 0.0)
        z = _bdot(h1, w2_ref[...]) + b2_ref[...]
        z_ref[...] = z
        blk = jnp.sum(z, axis=0, keepdims=True) * _mask8(hdim)

        @pl.when(i == 0)
        def _():
            zs_ref[...] = jnp.zeros_like(zs_ref)

        zs_ref[...] += blk

    specs = [pl.BlockSpec((bm, d), lambda i: (i, 0)),
             pl.BlockSpec((d, hdim), lambda i: (0, 0)),
             pl.BlockSpec((1, hdim), lambda i: (0, 0)),
             pl.BlockSpec((1, hdim), lambda i: (0, 0)),
             pl.BlockSpec((1, hdim), lambda i: (0, 0)),
             pl.BlockSpec((hdim, hdim), lambda i: (0, 0)),
             pl.BlockSpec((1, hdim), lambda i: (0, 0)),
             pl.BlockSpec((8, hdim), lambda i: (0, 0)),
             pl.BlockSpec((8, hdim), lambda i: (0, 0))]
    return pl.pallas_call(
        body,
        grid=(n // bm,),
        in_specs=specs,
        out_specs=[
            pl.BlockSpec((bm, hdim), lambda i: (i, 0)),
            pl.BlockSpec((8, hdim), lambda i: (0, 0)),
        ],
        out_shape=[
            jax.ShapeDtypeStruct((n, hdim), jnp.float32),
            jax.ShapeDtypeStruct((8, hdim), jnp.float32),
        ],
    )(pooled, w1, b1, g1, be1, w2, b2, ysum, ysumsq)


def _colsumsq_z(z, zsum):
    n, hdim = z.shape
    bm = _pick_rows(n, 2000)
    c = 1.0 / n

    def body(z_ref, s1_ref, s_ref):
        i = pl.program_id(0)
        mean = s1_ref[0:1, :] * c
        dd = z_ref[...] - mean
        blk = jnp.sum(dd * dd, axis=0, keepdims=True) * _mask8(hdim)

        @pl.when(i == 0)
        def _():
            s_ref[...] = jnp.zeros_like(s_ref)

        s_ref[...] += blk

    return pl.pallas_call(
        body,
        grid=(n // bm,),
        in_specs=[pl.BlockSpec((bm, hdim), lambda i: (i, 0)),
                  pl.BlockSpec((8, hdim), lambda i: (0, 0))],
        out_specs=pl.BlockSpec((8, hdim), lambda i: (0, 0)),
        out_shape=jax.ShapeDtypeStruct((8, hdim), jnp.float32),
    )(z, zsum)


def _bn_relu_out(z, zsum, zsumsq, g, b, out_dtype):
    n, hdim = z.shape
    bm = _pick_rows(n, 2000)
    c = 1.0 / n

    def body(z_ref, s1_ref, s2_ref, g_ref, b_ref, o_ref):
        mean = s1_ref[0:1, :] * c
        den = jnp.sqrt(s2_ref[0:1, :] * c + 1e-5)
        o_ref[...] = jnp.maximum(
            g_ref[...] * (z_ref[...] - mean) / den + b_ref[...],
            0.0).astype(o_ref.dtype)

    return pl.pallas_call(
        body,
        grid=(n // bm,),
        in_specs=[pl.BlockSpec((bm, hdim), lambda i: (i, 0)),
                  pl.BlockSpec((8, hdim), lambda i: (0, 0)),
                  pl.BlockSpec((8, hdim), lambda i: (0, 0)),
                  pl.BlockSpec((1, hdim), lambda i: (0, 0)),
                  pl.BlockSpec((1, hdim), lambda i: (0, 0))],
        out_specs=pl.BlockSpec((bm, hdim), lambda i: (i, 0)),
        out_shape=jax.ShapeDtypeStruct((n, hdim), out_dtype),
    )(z, zsum, zsumsq, g, b)


def _pool_body(gp_ref, h_ref, o_ref):
    o_ref[...] = _bdot(gp_ref[...], h_ref[...])


def _graph_pool_mm(graph_pool, h):
    g, n = graph_pool.shape
    d = h.shape[1]
    return pl.pallas_call(
        _pool_body,
        out_shape=jax.ShapeDtypeStruct((g, d), jnp.float32),
    )(graph_pool, h)


def _row(v):
    return v.reshape(1, -1)


def kernel(x, graph_pool, adj, params):
    h = x.astype(jnp.bfloat16)
    n_layers = len(params)
    for li, p in enumerate(params):
        pooled = _adj_matmul(adj, h)
        w1, b1 = p['W1'], _row(p['b1'])
        ysum = _colsum_y(pooled, w1, b1)
        ysumsq = _colsumsq_y(pooled, w1, b1, ysum)
        z, zsum = _apply_and_z(pooled, w1, b1, _row(p['g1']), _row(p['be1']),
                               p['W2'], _row(p['b2']), ysum, ysumsq)
        zsumsq = _colsumsq_z(z, zsum)
        last = li == n_layers - 1
        h = _bn_relu_out(z, zsum, zsumsq, _row(p['bn_g']), _row(p['bn_b']),
                         jnp.float32 if last else jnp.bfloat16)
    h_nodes = h
    pooled_h = _graph_pool_mm(graph_pool, h_nodes)
    return (pooled_h, h_nodes)
